# Initial kernel scaffold; baseline (speedup 1.0000x reference)
import jax
import jax.numpy as jnp
from jax.experimental import pallas as pl


def kernel(x, table, W, b):
    raise NotImplementedError("write your pallas kernel here")


# speedup vs baseline: 1.4751x; 1.4751x over previous
import functools

import jax
import jax.numpy as jnp
from jax import lax
from jax.experimental import pallas as pl
from jax.experimental.pallas import tpu as pltpu
from jax.experimental.pallas import tpu_sc as plsc

D = 64
OUT_D = 256
VPAD = 1024

NC = 2
NS = 16
NW = NC * NS
CHUNK = 128


def _proj_body(t_ref, w_ref, b_ref, o_ref):
    o_ref[...] = (
        jnp.dot(t_ref[...], w_ref[...], preferred_element_type=jnp.float32)
        + b_ref[...]
    )


def _project(table_pad, wt, b_row):
    return pl.pallas_call(
        _proj_body,
        out_shape=jax.ShapeDtypeStruct((VPAD, OUT_D), jnp.float32),
    )(table_pad, wt, b_row)


def _make_gather(bl):
    per_w = bl // NW
    n_chunk = per_w // CHUNK
    mesh = plsc.VectorSubcoreMesh(core_axis_name="c", subcore_axis_name="s")

    @functools.partial(
        pl.kernel,
        mesh=mesh,
        out_type=jax.ShapeDtypeStruct((bl, OUT_D), jnp.float32),
        scratch_types=[
            pltpu.VMEM((CHUNK,), jnp.int32),
            pltpu.VMEM((CHUNK, OUT_D), jnp.float32),
            pltpu.SemaphoreType.DMA,
        ],
    )
    def gather(ptable_hbm, idx_hbm, out_hbm, idx_v, rows_v, sem):
        wid = lax.axis_index("s") * NC + lax.axis_index("c")
        base = wid * per_w

        def body(i, carry):
            off = pl.multiple_of(base + i * CHUNK, CHUNK)
            pltpu.sync_copy(idx_hbm.at[pl.ds(off, CHUNK)], idx_v)
            pltpu.async_copy(ptable_hbm.at[idx_v], rows_v, sem).wait()
            pltpu.sync_copy(rows_v, out_hbm.at[pl.ds(off, CHUNK)])
            return carry

        lax.fori_loop(0, n_chunk, body, 0)

    return gather


def kernel(x, table, W, b):
    bsz, seq = x.shape
    bl = bsz * seq
    idx = x.reshape(bl).astype(jnp.int32)
    table_pad = jnp.zeros((VPAD, D), jnp.float32).at[: table.shape[0]].set(table)
    ptable = _project(table_pad, W.T, b.reshape(1, OUT_D))
    out = _make_gather(bl)(ptable, idx)
    return out.reshape(bsz, seq, OUT_D)


# speedup vs baseline: 1.5234x; 1.0328x over previous
import functools

import jax
import jax.numpy as jnp
from jax import lax
from jax.experimental import pallas as pl
from jax.experimental.pallas import tpu as pltpu
from jax.experimental.pallas import tpu_sc as plsc

D = 64
OUT_D = 256
VPAD = 1024

NC = 2
NS = 16
NW = NC * NS
CHUNK = 128


def _proj_body(t_ref, w_ref, b_ref, o_ref):
    o_ref[...] = (
        jnp.dot(t_ref[...], w_ref[...], preferred_element_type=jnp.float32)
        + b_ref[...]
    )


def _project(table_pad, wt, b_row):
    return pl.pallas_call(
        _proj_body,
        out_shape=jax.ShapeDtypeStruct((VPAD, OUT_D), jnp.float32),
    )(table_pad, wt, b_row)


NBUF = 3


def _make_gather(bl):
    per_w = bl // NW
    n_chunk = per_w // CHUNK
    mesh = plsc.VectorSubcoreMesh(core_axis_name="c", subcore_axis_name="s")

    @functools.partial(
        pl.kernel,
        mesh=mesh,
        out_type=jax.ShapeDtypeStruct((bl, OUT_D), jnp.float32),
        scratch_types=[
            pltpu.VMEM((per_w,), jnp.int32),
            pltpu.VMEM((NBUF, CHUNK, OUT_D), jnp.float32),
            pltpu.SemaphoreType.DMA((NBUF,)),
            pltpu.SemaphoreType.DMA((NBUF,)),
        ],
    )
    def gather(ptable_hbm, idx_hbm, out_hbm, idx_v, rows_v, gsem, ssem):
        wid = lax.axis_index("s") * NC + lax.axis_index("c")
        base = wid * per_w
        pltpu.sync_copy(idx_hbm.at[pl.ds(base, per_w)], idx_v)

        def fire_gather(c):
            s = c % NBUF
            return pltpu.async_copy(
                ptable_hbm.at[idx_v.at[pl.ds(c * CHUNK, CHUNK)]],
                rows_v.at[s],
                gsem.at[s],
            )

        def fire_store(c):
            s = c % NBUF
            return pltpu.async_copy(
                rows_v.at[s],
                out_hbm.at[pl.ds(base + c * CHUNK, CHUNK)],
                ssem.at[s],
            )

        gh = {c: fire_gather(c) for c in range(min(1, n_chunk))}
        sh = {}
        for i in range(n_chunk):
            nxt = i + 1
            if nxt < n_chunk:
                prev = nxt - NBUF
                if prev >= 0:
                    sh[prev].wait()
                gh[nxt] = fire_gather(nxt)
            gh[i].wait()
            sh[i] = fire_store(i)
        for c in range(max(0, n_chunk - NBUF), n_chunk):
            sh[c].wait()

    return gather


def kernel(x, table, W, b):
    bsz, seq = x.shape
    bl = bsz * seq
    idx = x.reshape(bl).astype(jnp.int32)
    table_pad = jnp.zeros((VPAD, D), jnp.float32).at[: table.shape[0]].set(table)
    ptable = _project(table_pad, W.T, b.reshape(1, OUT_D))
    out = _make_gather(bl)(ptable, idx)
    return out.reshape(bsz, seq, OUT_D)


# speedup vs baseline: 2.0985x; 1.3775x over previous
import functools

import jax
import jax.numpy as jnp
from jax import lax
from jax.experimental import pallas as pl
from jax.experimental.pallas import tpu as pltpu
from jax.experimental.pallas import tpu_sc as plsc

D = 64
DP = 128
OUT_D = 256

NC = 2
NS = 16
NW = NC * NS
CHUNK = 128
NBUF = 3


def _make_gather(bl):
    per_w = bl // NW
    n_chunk = per_w // CHUNK
    mesh = plsc.VectorSubcoreMesh(core_axis_name="c", subcore_axis_name="s")

    @functools.partial(
        pl.kernel,
        mesh=mesh,
        out_type=jax.ShapeDtypeStruct((bl, DP), jnp.float32),
        scratch_types=[
            pltpu.VMEM((per_w,), jnp.int32),
            pltpu.VMEM((NBUF, CHUNK, DP), jnp.float32),
            pltpu.SemaphoreType.DMA((NBUF,)),
            pltpu.SemaphoreType.DMA((NBUF,)),
        ],
    )
    def gather(table_hbm, idx_hbm, out_hbm, idx_v, rows_v, gsem, ssem):
        wid = lax.axis_index("s") * NC + lax.axis_index("c")
        base = wid * per_w
        pltpu.sync_copy(idx_hbm.at[pl.ds(base, per_w)], idx_v)

        def fire_gather(c):
            s = c % NBUF
            return pltpu.async_copy(
                table_hbm.at[idx_v.at[pl.ds(c * CHUNK, CHUNK)]],
                rows_v.at[s],
                gsem.at[s],
            )

        def fire_store(c):
            s = c % NBUF
            return pltpu.async_copy(
                rows_v.at[s],
                out_hbm.at[pl.ds(base + c * CHUNK, CHUNK)],
                ssem.at[s],
            )

        gh = {c: fire_gather(c) for c in range(min(1, n_chunk))}
        sh = {}
        for i in range(n_chunk):
            nxt = i + 1
            if nxt < n_chunk:
                prev = nxt - NBUF
                if prev >= 0:
                    sh[prev].wait()
                gh[nxt] = fire_gather(nxt)
            gh[i].wait()
            sh[i] = fire_store(i)
        for c in range(max(0, n_chunk - NBUF), n_chunk):
            sh[c].wait()

    return gather


def _mm_body(seq, a_ref, w_ref, b_ref, o_ref):
    for l in range(seq):
        o_ref[:, l, :] = (
            jnp.dot(a_ref[l], w_ref[...], preferred_element_type=jnp.float32)
            + b_ref[...]
        )


def _project(a, w2, b2, bsz, seq):
    s_blk = 512
    return pl.pallas_call(
        functools.partial(_mm_body, seq),
        grid=(bsz // s_blk,),
        in_specs=[
            pl.BlockSpec((seq, s_blk, DP), lambda i: (0, i, 0)),
            pl.BlockSpec((DP, OUT_D), lambda i: (0, 0)),
            pl.BlockSpec((1, OUT_D), lambda i: (0, 0)),
        ],
        out_specs=pl.BlockSpec((s_blk, seq, OUT_D), lambda i: (i, 0, 0)),
        out_shape=jax.ShapeDtypeStruct((bsz, seq, OUT_D), jnp.float32),
    )(a, w2, b2)


def kernel(x, table, W, b):
    bsz, seq = x.shape
    bl = bsz * seq
    idx = x.T.reshape(bl).astype(jnp.int32)
    table_p = jnp.pad(table, ((0, 0), (0, DP - D)))
    g = _make_gather(bl)(table_p, idx)
    a = g.reshape(seq, bsz, DP)
    w2 = jnp.pad(W.T, ((0, DP - D), (0, 0)))
    b2 = b.reshape(1, OUT_D)
    return _project(a, w2, b2, bsz, seq)


# speedup vs baseline: 2.1003x; 1.0009x over previous
import functools

import jax
import jax.numpy as jnp
from jax import lax
from jax.experimental import pallas as pl
from jax.experimental.pallas import tpu as pltpu
from jax.experimental.pallas import tpu_sc as plsc

D = 64
DP = 128
OUT_D = 256

NC = 2
NS = 16
NW = NC * NS
CHUNK = 128
NBUF = 4
AHEAD = 2


def _make_gather(bl):
    per_w = bl // NW
    n_chunk = per_w // CHUNK
    mesh = plsc.VectorSubcoreMesh(core_axis_name="c", subcore_axis_name="s")

    @functools.partial(
        pl.kernel,
        mesh=mesh,
        out_type=jax.ShapeDtypeStruct((bl, DP), jnp.float32),
        scratch_types=[
            pltpu.VMEM((per_w,), jnp.int32),
            pltpu.VMEM((NBUF, CHUNK, DP), jnp.float32),
            pltpu.SemaphoreType.DMA((NBUF,)),
            pltpu.SemaphoreType.DMA((NBUF,)),
        ],
    )
    def gather(table_hbm, idx_hbm, out_hbm, idx_v, rows_v, gsem, ssem):
        wid = lax.axis_index("s") * NC + lax.axis_index("c")
        base = wid * per_w
        pltpu.sync_copy(idx_hbm.at[pl.ds(base, per_w)], idx_v)

        def fire_gather(c):
            s = c % NBUF
            return pltpu.async_copy(
                table_hbm.at[idx_v.at[pl.ds(c * CHUNK, CHUNK)]],
                rows_v.at[s],
                gsem.at[s],
            )

        def fire_store(c):
            s = c % NBUF
            return pltpu.async_copy(
                rows_v.at[s],
                out_hbm.at[pl.ds(base + c * CHUNK, CHUNK)],
                ssem.at[s],
            )

        gh = {c: fire_gather(c) for c in range(min(AHEAD, n_chunk))}
        sh = {}
        for i in range(n_chunk):
            nxt = i + AHEAD
            if nxt < n_chunk:
                prev = nxt - NBUF
                if prev >= 0:
                    sh[prev].wait()
                gh[nxt] = fire_gather(nxt)
            gh[i].wait()
            sh[i] = fire_store(i)
        drain_from = max(0, n_chunk - NBUF) if n_chunk > AHEAD else 0
        for c in range(drain_from, n_chunk):
            sh[c].wait()

    return gather


def _mm_body(seq, a_ref, w_ref, b_ref, o_ref):
    for l in range(seq):
        o_ref[:, l, :] = (
            jnp.dot(a_ref[l], w_ref[...], preferred_element_type=jnp.float32)
            + b_ref[...]
        )


def _project(a, w2, b2, bsz, seq):
    s_blk = 512
    return pl.pallas_call(
        functools.partial(_mm_body, seq),
        grid=(bsz // s_blk,),
        in_specs=[
            pl.BlockSpec((seq, s_blk, DP), lambda i: (0, i, 0)),
            pl.BlockSpec((DP, OUT_D), lambda i: (0, 0)),
            pl.BlockSpec((1, OUT_D), lambda i: (0, 0)),
        ],
        out_specs=pl.BlockSpec((s_blk, seq, OUT_D), lambda i: (i, 0, 0)),
        out_shape=jax.ShapeDtypeStruct((bsz, seq, OUT_D), jnp.float32),
    )(a, w2, b2)


def kernel(x, table, W, b):
    bsz, seq = x.shape
    bl = bsz * seq
    idx = x.T.reshape(bl).astype(jnp.int32)
    table_p = jnp.pad(table, ((0, 0), (0, DP - D)))
    g = _make_gather(bl)(table_p, idx)
    a = g.reshape(seq, bsz, DP)
    w2 = jnp.pad(W.T, ((0, DP - D), (0, 0)))
    b2 = b.reshape(1, OUT_D)
    return _project(a, w2, b2, bsz, seq)
